# baseline (speedup 1.0000x reference)
import jax
import jax.numpy as jnp
from jax.experimental import pallas as pl

INPUT_DIM = 2048
NUM_EXPERTS = 16
TOP_K = 2
NUM_TOKENS = 16384

BLK = 1024
NSTEP = NUM_TOKENS // (2 * BLK)


def _top2(logits):
    lanes = jax.lax.broadcasted_iota(jnp.int32, logits.shape, 1)
    l1 = jnp.max(logits, axis=1, keepdims=True)
    i1 = jnp.min(jnp.where(logits == l1, lanes, NUM_EXPERTS), axis=1, keepdims=True)
    l_masked = jnp.where(lanes == i1, -jnp.inf, logits)
    l2 = jnp.max(l_masked, axis=1, keepdims=True)
    i2 = jnp.min(
        jnp.where(l_masked == l2, lanes, NUM_EXPERTS), axis=1, keepdims=True
    )
    s = jnp.sum(jnp.exp(logits - l1), axis=1, keepdims=True)
    w2 = jnp.exp(l2 - l1)
    return (
        jnp.concatenate([jnp.ones_like(w2), w2], axis=1) / s,
        jnp.concatenate([i1, i2], axis=1),
    )


def _gating_kernel(xa_ref, xb_ref, wt_ref, b_ref, wa_ref, ia_ref, wb_ref, ib_ref):
    wt = wt_ref[...]
    bias = b_ref[...]
    la = jnp.dot(xa_ref[0], wt, preferred_element_type=jnp.float32) + bias
    wa_ref[0], ia_ref[0] = _top2(la)
    lb = jnp.dot(xb_ref[0], wt, preferred_element_type=jnp.float32) + bias
    wb_ref[0], ib_ref[0] = _top2(lb)


@jax.jit
def kernel(x, W, b):
    wt = W.T
    b2 = b.reshape(1, NUM_EXPERTS)
    x4 = x.reshape(NSTEP * 2, BLK, INPUT_DIM)
    wa, ia, wb, ib = pl.pallas_call(
        _gating_kernel,
        grid=(NSTEP,),
        in_specs=[
            pl.BlockSpec((1, BLK, INPUT_DIM), lambda i: (2 * i, 0, 0)),
            pl.BlockSpec((1, BLK, INPUT_DIM), lambda i: (2 * i + 1, 0, 0)),
            pl.BlockSpec((INPUT_DIM, NUM_EXPERTS), lambda i: (0, 0)),
            pl.BlockSpec((1, NUM_EXPERTS), lambda i: (0, 0)),
        ],
        out_specs=[
            pl.BlockSpec((1, BLK, TOP_K), lambda i: (i, 0, 0)),
            pl.BlockSpec((1, BLK, TOP_K), lambda i: (i, 0, 0)),
            pl.BlockSpec((1, BLK, TOP_K), lambda i: (i, 0, 0)),
            pl.BlockSpec((1, BLK, TOP_K), lambda i: (i, 0, 0)),
        ],
        out_shape=[
            jax.ShapeDtypeStruct((NSTEP, BLK, TOP_K), jnp.float32),
            jax.ShapeDtypeStruct((NSTEP, BLK, TOP_K), jnp.int32),
            jax.ShapeDtypeStruct((NSTEP, BLK, TOP_K), jnp.float32),
            jax.ShapeDtypeStruct((NSTEP, BLK, TOP_K), jnp.int32),
        ],
    )(x4, x4, wt, b2)
    w_out = jnp.stack([wa, wb], axis=1).reshape(NUM_TOKENS, TOP_K)
    i_out = jnp.stack([ia, ib], axis=1).reshape(NUM_TOKENS, TOP_K)
    return (w_out, i_out)
